# baseline (speedup 1.0000x reference)
import jax
import jax.numpy as jnp
from jax.experimental import pallas as pl

_INV_CELL = 64.0
_N = 1_000_000
_R = 2000


def _body(pos_ref, sf_ref, gf_ref):
    pos = pos_ref[...]
    rel = pos * _INV_CELL
    base = rel.astype(jnp.int32).astype(jnp.float32)
    frac = rel - base
    fx = frac[:, 0:1]
    fy = frac[:, 1:2]
    fz = frac[:, 2:3]

    r = pos.shape[0]

    wi = jax.lax.broadcasted_iota(jnp.int32, (r, 8), 1)
    i8 = (wi >> 2) & 1
    j8 = (wi >> 1) & 1
    k8 = wi & 1
    bx8 = jnp.where(i8 == 1, fx, 1.0 - fx)
    by8 = jnp.where(j8 == 1, fy, 1.0 - fy)
    bz8 = jnp.where(k8 == 1, fz, 1.0 - fz)
    sf_ref[...] = bx8 * by8 * bz8

    li = jax.lax.broadcasted_iota(jnp.int32, (r, 24), 1)
    w24 = li // 3
    d24 = li - 3 * w24
    i24 = (w24 >> 2) & 1
    j24 = (w24 >> 1) & 1
    k24 = w24 & 1
    sx = jnp.sign(fx) * _INV_CELL
    sy = jnp.sign(fy) * _INV_CELL
    sz = jnp.sign(fz) * _INV_CELL
    bx = jnp.where(i24 == 1, fx, 1.0 - fx)
    by = jnp.where(j24 == 1, fy, 1.0 - fy)
    bz = jnp.where(k24 == 1, fz, 1.0 - fz)
    dbx = jnp.where(i24 == 1, sx, -sx)
    dby = jnp.where(j24 == 1, sy, -sy)
    dbz = jnp.where(k24 == 1, sz, -sz)
    tx = jnp.where(d24 == 0, dbx, bx)
    ty = jnp.where(d24 == 1, dby, by)
    tz = jnp.where(d24 == 2, dbz, bz)
    gf_ref[...] = tx * ty * tz


def kernel(position_stack):
    n = position_stack.shape[0]
    grid = (n // _R,)
    sf, gf = pl.pallas_call(
        _body,
        grid=grid,
        in_specs=[pl.BlockSpec((_R, 3), lambda i: (i, 0))],
        out_specs=[
            pl.BlockSpec((_R, 8), lambda i: (i, 0)),
            pl.BlockSpec((_R, 24), lambda i: (i, 0)),
        ],
        out_shape=[
            jax.ShapeDtypeStruct((n, 8), jnp.float32),
            jax.ShapeDtypeStruct((n, 24), jnp.float32),
        ],
    )(position_stack)
    return sf, gf.reshape(n, 8, 3)
